# baseline (speedup 1.0000x reference)
import functools

import jax
import jax.numpy as jnp
from jax import lax
from jax.experimental import pallas as pl
from jax.experimental.pallas import tpu as pltpu
from jax.experimental.pallas import tpu_sc as plsc

N_NODES = 10000
N_EDGES = 320000
EMB = 128
HALF = N_NODES // 2

NC, NS = 2, 16
NW = NC * NS
K = 32
NBUF = 2
E_PAD = 327680
TOT_B = E_PAD // K
NBLK_T = TOT_B // NBUF
EALIGN = K * NS * NBUF
AGG_ROWS = 5120
DUMP_ROW = 5100

KX = 128
XBT = 3
X_PAD = NW * XBT * KX

_mesh = plsc.VectorSubcoreMesh(
    core_axis_name="c", subcore_axis_name="s", num_cores=NC, num_subcores=NS)


@functools.partial(
    pl.kernel,
    out_type=jax.ShapeDtypeStruct((X_PAD, EMB), jnp.float32),
    mesh=_mesh,
    scratch_types=[
        pltpu.VMEM((XBT, KX), jnp.int32),
        pltpu.VMEM((KX, EMB), jnp.float32),
    ],
)
def _embed_sc(table_hbm, idx_hbm, out_hbm, idx_v, rows_v):
    cid = lax.axis_index("c")
    sid = lax.axis_index("s")
    wid = cid * NS + sid
    pltpu.sync_copy(idx_hbm.at[wid], idx_v)

    def body(b, carry):
        pltpu.sync_copy(table_hbm.at[idx_v.at[b]], rows_v)
        pltpu.sync_copy(rows_v, out_hbm.at[pl.ds(wid * XBT * KX + b * KX, KX)])
        return carry

    lax.fori_loop(0, XBT, body, 0)


@functools.partial(
    pl.kernel,
    out_type=jax.ShapeDtypeStruct((NC, AGG_ROWS, EMB), jnp.float32),
    mesh=_mesh,
    scratch_types=[
        pltpu.VMEM((2, NBUF, K), jnp.int32),
        pltpu.VMEM((2, NBUF, K), jnp.int32),
        pltpu.VMEM((NBUF, K, EMB), jnp.float32),
        pltpu.VMEM((16,), jnp.int32),
        pltpu.VMEM_SHARED((N_NODES, EMB), jnp.float32),
        pltpu.VMEM_SHARED((AGG_ROWS, EMB), jnp.float32),
        pltpu.SemaphoreType.DMA((NBUF,)),
        pltpu.SemaphoreType.DMA((NBUF,)),
        pltpu.SemaphoreType.DMA,
    ],
)
def _segment_sc(h_hbm, src_hbm, dst_hbm, meta_hbm, out_hbm,
                sidx_v, didx_v, rows_v, meta_v, h_sp, agg_sh,
                gsem, ssem, isem):
    cid = lax.axis_index("c")
    sid = lax.axis_index("s")

    @pl.when(sid < NS - 1)
    def _stage_main():
        pltpu.sync_copy(h_hbm.at[pl.ds(sid * 640, 640)],
                        h_sp.at[pl.ds(sid * 640, 640)])

    @pl.when(sid == NS - 1)
    def _stage_tail():
        pltpu.sync_copy(h_hbm.at[pl.ds(9600, 400)],
                        h_sp.at[pl.ds(9600, 400)])

    zv = jnp.zeros((16,), jnp.float32)

    def zrow(r, carry):
        def zcol(ci, carry2):
            rows_v[0, r, pl.ds(ci * 16, 16)] = zv
            return carry2
        return lax.fori_loop(0, EMB // 16, zcol, carry)

    lax.fori_loop(0, K, zrow, 0)
    zshare = AGG_ROWS // NS // K

    def zs(j, carry):
        pltpu.sync_copy(rows_v.at[0],
                        agg_sh.at[pl.ds((sid * zshare + j) * K, K)])
        return carry

    lax.fori_loop(0, zshare, zs, 0)

    pltpu.sync_copy(meta_hbm, meta_v)
    nblk0 = meta_v[...][0]
    nblk_me = jnp.where(cid == 0, nblk0 // NS, (NBLK_T - nblk0) // NS)
    start = jnp.where(cid == 0, sid * nblk_me, nblk0 + sid * nblk_me)

    @pl.when(nblk_me > 0)
    def _prime():
        pltpu.sync_copy(src_hbm.at[start], sidx_v.at[0])
        pltpu.sync_copy(dst_hbm.at[start], didx_v.at[0])

    plsc.subcore_barrier()

    def block(blk, carry):
        p = blk % 2
        row = start + blk

        @pl.when(blk + 1 < nblk_me)
        def _prefetch_idx():
            pltpu.async_copy(src_hbm.at[row + 1], sidx_v.at[1 - p], isem)
            pltpu.async_copy(dst_hbm.at[row + 1], didx_v.at[1 - p], isem)

        @pl.when(blk > 0)
        def _wait_idx():
            pltpu.make_async_copy(src_hbm.at[row], sidx_v.at[p], isem).wait()
            pltpu.make_async_copy(dst_hbm.at[row], didx_v.at[p], isem).wait()

        gds = []
        for j in range(NBUF):
            @pl.when(blk > 0)
            def _wait_prev_scatter():
                pltpu.make_async_copy(
                    rows_v.at[j], agg_sh.at[didx_v.at[1 - p, j]],
                    ssem.at[j]).wait()

            gds.append(pltpu.async_copy(
                h_sp.at[sidx_v.at[p, j]], rows_v.at[j], gsem.at[j]))
        for j in range(NBUF):
            gds[j].wait()
            pltpu.async_copy(
                rows_v.at[j], agg_sh.at[didx_v.at[p, j]], ssem.at[j], add=True)
        return carry

    lax.fori_loop(0, nblk_me, block, 0)

    @pl.when(nblk_me > 0)
    def _drain():
        lastp = (nblk_me - 1) % 2
        for j in range(NBUF):
            pltpu.make_async_copy(
                rows_v.at[j], agg_sh.at[didx_v.at[lastp, j]],
                ssem.at[j]).wait()

    plsc.subcore_barrier()

    share = AGG_ROWS // NS
    pltpu.sync_copy(agg_sh.at[pl.ds(sid * share, share)],
                    out_hbm.at[cid, pl.ds(sid * share, share)])


def _mlp_body(relu_out, h_ref, a_ref, w1_ref, b1_ref, w2_ref, b2_ref, o_ref):
    z = h_ref[...] + a_ref[0]
    z = jnp.dot(z, w1_ref[...], preferred_element_type=jnp.float32) + b1_ref[...]
    z = jnp.maximum(z, 0.0)
    o = jnp.dot(z, w2_ref[...], preferred_element_type=jnp.float32) + b2_ref[...]
    if relu_out:
        o = jnp.maximum(o, 0.0)
    o_ref[...] = o


_CHUNK = 1000
_CPH = HALF // _CHUNK


def _mlp(h, agg, W1, b1, W2, b2, relu_out):
    bs_h = pl.BlockSpec((_CHUNK, EMB), lambda i: (i, 0))
    bs_a = pl.BlockSpec((1, _CHUNK, EMB), lambda i: (i // _CPH, i % _CPH, 0))
    bs_w = pl.BlockSpec((EMB, EMB), lambda i: (0, 0))
    bs_b = pl.BlockSpec((1, EMB), lambda i: (0, 0))
    return pl.pallas_call(
        functools.partial(_mlp_body, relu_out),
        grid=(N_NODES // _CHUNK,),
        in_specs=[bs_h, bs_a, bs_w, bs_b, bs_w, bs_b],
        out_specs=bs_h,
        out_shape=jax.ShapeDtypeStruct((N_NODES, EMB), jnp.float32),
    )(h, agg, W1, b1.reshape(1, EMB), W2, b2.reshape(1, EMB))


def kernel(x, edge_index, emb_table,
           W1_0, b1_0, W2_0, b2_0,
           W1_1, b1_1, W2_1, b2_1,
           W1_2, b1_2, W2_2, b2_2,
           W1_3, b1_3, W2_3, b2_3,
           W1_4, b1_4, W2_4, b2_4):
    src = edge_index[0]
    dst = edge_index[1]

    in1 = (dst >= HALF).astype(jnp.int32)
    c0 = jnp.cumsum(1 - in1)
    b_cnt = c0[-1]
    b_pad = ((b_cnt + EALIGN - 1) // EALIGN) * EALIGN
    c1 = jnp.cumsum(in1)
    pos = jnp.where(in1 == 1, b_pad + c1 - 1, c0 - 1)
    dst_loc = jnp.where(in1 == 1, dst - HALF, dst)
    src_s = jnp.zeros((E_PAD,), jnp.int32).at[pos].set(src)
    dst_s = jnp.full((E_PAD,), DUMP_ROW, jnp.int32).at[pos].set(dst_loc)
    src_p = src_s.reshape(NBLK_T, NBUF, K)
    dst_p = dst_s.reshape(NBLK_T, NBUF, K)
    meta = jnp.full((16,), b_pad // (K * NBUF), jnp.int32)

    x_p = jnp.concatenate(
        [x[:, 0], jnp.zeros((X_PAD - N_NODES,), jnp.int32)]).reshape(NW, XBT, KX)
    h = _embed_sc(emb_table, x_p)[:N_NODES]

    params = [
        (W1_0, b1_0, W2_0, b2_0),
        (W1_1, b1_1, W2_1, b2_1),
        (W1_2, b1_2, W2_2, b2_2),
        (W1_3, b1_3, W2_3, b2_3),
        (W1_4, b1_4, W2_4, b2_4),
    ]
    for i, (W1, b1, W2, b2) in enumerate(params):
        agg = _segment_sc(h, src_p, dst_p, meta)
        h = _mlp(h, agg, W1, b1, W2, b2, relu_out=(i < 4))
    return h


# speedup vs baseline: 2.0560x; 2.0560x over previous
import functools

import jax
import jax.numpy as jnp
from jax import lax
from jax.experimental import pallas as pl
from jax.experimental.pallas import tpu as pltpu
from jax.experimental.pallas import tpu_sc as plsc

N_NODES = 10000
N_EDGES = 320000
EMB = 128
HALF = N_NODES // 2

NC, NS = 2, 16
NW = NC * NS
K = 32
NBUF = 2
E_PAD = 327680
TOT_B = E_PAD // K
NBLK_T = TOT_B // NBUF
NBLK_ME = NBLK_T // NS
AGG_ROWS = 5120
DUMP_ROW = 5100

KX = 128
XBT = 3
X_PAD = NW * XBT * KX

_mesh = plsc.VectorSubcoreMesh(
    core_axis_name="c", subcore_axis_name="s", num_cores=NC, num_subcores=NS)


@functools.partial(
    pl.kernel,
    out_type=jax.ShapeDtypeStruct((X_PAD, EMB), jnp.float32),
    mesh=_mesh,
    scratch_types=[
        pltpu.VMEM((XBT, KX), jnp.int32),
        pltpu.VMEM((KX, EMB), jnp.float32),
    ],
)
def _embed_sc(table_hbm, idx_hbm, out_hbm, idx_v, rows_v):
    cid = lax.axis_index("c")
    sid = lax.axis_index("s")
    wid = cid * NS + sid
    pltpu.sync_copy(idx_hbm.at[wid], idx_v)

    def body(b, carry):
        pltpu.sync_copy(table_hbm.at[idx_v.at[b]], rows_v)
        pltpu.sync_copy(rows_v, out_hbm.at[pl.ds(wid * XBT * KX + b * KX, KX)])
        return carry

    lax.fori_loop(0, XBT, body, 0)


@functools.partial(
    pl.kernel,
    out_type=jax.ShapeDtypeStruct((NC, AGG_ROWS, EMB), jnp.float32),
    mesh=_mesh,
    scratch_types=[
        pltpu.VMEM((2, NBUF, K), jnp.int32),
        pltpu.VMEM((2, NBUF, K), jnp.int32),
        pltpu.VMEM((NBUF, K, EMB), jnp.float32),
        pltpu.VMEM_SHARED((N_NODES, EMB), jnp.float32),
        pltpu.VMEM_SHARED((AGG_ROWS, EMB), jnp.float32),
        pltpu.SemaphoreType.DMA((NBUF,)),
        pltpu.SemaphoreType.DMA((NBUF,)),
        pltpu.SemaphoreType.DMA,
    ],
)
def _segment_sc(h_hbm, src_hbm, dst_hbm, out_hbm,
                sidx_v, didx_v, rows_v, h_sp, agg_sh,
                gsem, ssem, isem):
    cid = lax.axis_index("c")
    sid = lax.axis_index("s")

    @pl.when(sid < NS - 1)
    def _stage_main():
        pltpu.sync_copy(h_hbm.at[pl.ds(sid * 640, 640)],
                        h_sp.at[pl.ds(sid * 640, 640)])

    @pl.when(sid == NS - 1)
    def _stage_tail():
        pltpu.sync_copy(h_hbm.at[pl.ds(9600, 400)],
                        h_sp.at[pl.ds(9600, 400)])

    zv = jnp.zeros((16,), jnp.float32)

    def zrow(r, carry):
        def zcol(ci, carry2):
            rows_v[0, r, pl.ds(ci * 16, 16)] = zv
            return carry2
        return lax.fori_loop(0, EMB // 16, zcol, carry)

    lax.fori_loop(0, K, zrow, 0)
    zshare = AGG_ROWS // NS // K

    def zs(j, carry):
        pltpu.sync_copy(rows_v.at[0],
                        agg_sh.at[pl.ds((sid * zshare + j) * K, K)])
        return carry

    lax.fori_loop(0, zshare, zs, 0)

    start = sid * NBLK_ME

    pltpu.sync_copy(src_hbm.at[start], sidx_v.at[0])
    pltpu.sync_copy(dst_hbm.at[cid, start], didx_v.at[0])

    plsc.subcore_barrier()

    def block(blk, carry):
        p = blk % 2
        row = start + blk

        @pl.when(blk + 1 < NBLK_ME)
        def _prefetch_idx():
            pltpu.async_copy(src_hbm.at[row + 1], sidx_v.at[1 - p], isem)
            pltpu.async_copy(dst_hbm.at[cid, row + 1], didx_v.at[1 - p], isem)

        @pl.when(blk > 0)
        def _wait_idx():
            pltpu.make_async_copy(src_hbm.at[row], sidx_v.at[p], isem).wait()
            pltpu.make_async_copy(dst_hbm.at[cid, row], didx_v.at[p],
                                  isem).wait()

        for j in range(NBUF):
            for ch in range(K // 16):
                dv = didx_v[p, j, pl.ds(ch * 16, 16)]
                didx_v[p, j, pl.ds(ch * 16, 16)] = jnp.where(
                    dv >= DUMP_ROW, DUMP_ROW + sid, dv)

        gds = []
        for j in range(NBUF):
            @pl.when(blk > 0)
            def _wait_prev_scatter():
                pltpu.make_async_copy(
                    rows_v.at[j], agg_sh.at[didx_v.at[1 - p, j]],
                    ssem.at[j]).wait()

            gds.append(pltpu.async_copy(
                h_sp.at[sidx_v.at[p, j]], rows_v.at[j], gsem.at[j]))
        for j in range(NBUF):
            gds[j].wait()
            pltpu.async_copy(
                rows_v.at[j], agg_sh.at[didx_v.at[p, j]], ssem.at[j], add=True)
        return carry

    lax.fori_loop(0, NBLK_ME, block, 0)

    lastp = (NBLK_ME - 1) % 2
    for j in range(NBUF):
        pltpu.make_async_copy(
            rows_v.at[j], agg_sh.at[didx_v.at[lastp, j]],
            ssem.at[j]).wait()

    plsc.subcore_barrier()

    share = AGG_ROWS // NS
    pltpu.sync_copy(agg_sh.at[pl.ds(sid * share, share)],
                    out_hbm.at[cid, pl.ds(sid * share, share)])


def _mlp_body(relu_out, h_ref, a_ref, w1_ref, b1_ref, w2_ref, b2_ref, o_ref):
    z = h_ref[...] + a_ref[0]
    z = jnp.dot(z, w1_ref[...], preferred_element_type=jnp.float32) + b1_ref[...]
    z = jnp.maximum(z, 0.0)
    o = jnp.dot(z, w2_ref[...], preferred_element_type=jnp.float32) + b2_ref[...]
    if relu_out:
        o = jnp.maximum(o, 0.0)
    o_ref[...] = o


_CHUNK = 1000
_CPH = HALF // _CHUNK


def _mlp(h, agg, W1, b1, W2, b2, relu_out):
    bs_h = pl.BlockSpec((_CHUNK, EMB), lambda i: (i, 0))
    bs_a = pl.BlockSpec((1, _CHUNK, EMB), lambda i: (i // _CPH, i % _CPH, 0))
    bs_w = pl.BlockSpec((EMB, EMB), lambda i: (0, 0))
    bs_b = pl.BlockSpec((1, EMB), lambda i: (0, 0))
    return pl.pallas_call(
        functools.partial(_mlp_body, relu_out),
        grid=(N_NODES // _CHUNK,),
        in_specs=[bs_h, bs_a, bs_w, bs_b, bs_w, bs_b],
        out_specs=bs_h,
        out_shape=jax.ShapeDtypeStruct((N_NODES, EMB), jnp.float32),
    )(h, agg, W1, b1.reshape(1, EMB), W2, b2.reshape(1, EMB))


def kernel(x, edge_index, emb_table,
           W1_0, b1_0, W2_0, b2_0,
           W1_1, b1_1, W2_1, b2_1,
           W1_2, b1_2, W2_2, b2_2,
           W1_3, b1_3, W2_3, b2_3,
           W1_4, b1_4, W2_4, b2_4):
    src = edge_index[0]
    dst = edge_index[1]

    pad_e = E_PAD - N_EDGES
    src_s = jnp.concatenate([src, jnp.zeros((pad_e,), jnp.int32)])
    dstf = jnp.concatenate([dst, jnp.full((pad_e,), -1, jnp.int32)])
    dst0 = jnp.where((dstf >= 0) & (dstf < HALF), dstf, DUMP_ROW)
    dst1 = jnp.where(dstf >= HALF, dstf - HALF, DUMP_ROW)
    src_p = src_s.reshape(NBLK_T, NBUF, K)
    dst_p = jnp.stack([dst0, dst1]).reshape(NC, NBLK_T, NBUF, K)

    x_p = jnp.concatenate(
        [x[:, 0], jnp.zeros((X_PAD - N_NODES,), jnp.int32)]).reshape(NW, XBT, KX)
    h = _embed_sc(emb_table, x_p)[:N_NODES]

    params = [
        (W1_0, b1_0, W2_0, b2_0),
        (W1_1, b1_1, W2_1, b2_1),
        (W1_2, b1_2, W2_2, b2_2),
        (W1_3, b1_3, W2_3, b2_3),
        (W1_4, b1_4, W2_4, b2_4),
    ]
    for i, (W1, b1, W2, b2) in enumerate(params):
        agg = _segment_sc(h, src_p, dst_p)
        h = _mlp(h, agg, W1, b1, W2, b2, relu_out=(i < 4))
    return h
